# Initial kernel scaffold; baseline (speedup 1.0000x reference)
import jax
import jax.numpy as jnp
from jax.experimental import pallas as pl


def kernel(timesteps, pe, W1, b1, W2, b2):
    raise NotImplementedError("write your pallas kernel here")


# speedup vs baseline: 1.1318x; 1.1318x over previous
import functools

import jax
import jax.numpy as jnp
from jax import lax
from jax.experimental import pallas as pl
from jax.experimental.pallas import tpu as pltpu
from jax.experimental.pallas import tpu_sc as plsc

NC, NS, LANES = 2, 16, 16
NW = NC * NS
B = 4096
D = 512
T = 512
TBL = 1024
BPW = B // NW


def _mlp_table_body(pe_ref, w1_ref, b1_ref, w2_ref, b2_ref, out_ref):
    x = pe_ref[...]
    h = jnp.dot(x, w1_ref[...], preferred_element_type=jnp.float32) + b1_ref[...]
    h = h * jax.nn.sigmoid(h)
    out_ref[...] = (
        jnp.dot(h, w2_ref[...], preferred_element_type=jnp.float32) + b2_ref[...]
    )


@functools.cache
def _sc_gather():
    mesh = plsc.VectorSubcoreMesh(core_axis_name="c", subcore_axis_name="s")

    @functools.partial(
        pl.kernel,
        out_type=jax.ShapeDtypeStruct((B, T), jnp.float32),
        mesh=mesh,
        scratch_types=[
            pltpu.VMEM((BPW,), jnp.float32),
            pltpu.VMEM((BPW,), jnp.int32),
            pltpu.VMEM((BPW, T), jnp.float32),
            pltpu.SemaphoreType.DMA,
        ],
    )
    def body(ts_hbm, table_hbm, out_hbm, ts_v, idx_v, rows_v, sem):
        wid = lax.axis_index("s") * NC + lax.axis_index("c")
        base = wid * BPW
        pltpu.sync_copy(ts_hbm.at[pl.ds(base, BPW)], ts_v)
        for i in range(BPW // LANES):
            t = ts_v[pl.ds(i * LANES, LANES)]
            idx_v[pl.ds(i * LANES, LANES)] = (t * 1000.0).astype(jnp.int32)
        pltpu.async_copy(table_hbm.at[idx_v], rows_v, sem).wait()
        pltpu.sync_copy(rows_v, out_hbm.at[pl.ds(base, BPW)])

    return body


@jax.jit
def kernel(timesteps, pe, W1, b1, W2, b2):
    pe2 = pe[:TBL, 0, :]
    table = pl.pallas_call(
        _mlp_table_body,
        out_shape=jax.ShapeDtypeStruct((TBL, T), jnp.float32),
    )(pe2, W1, b1.reshape(1, T), W2, b2.reshape(1, T))
    return _sc_gather()(timesteps, table)
